# baseline (speedup 1.0000x reference)
import jax
import jax.numpy as jnp
from jax import lax
from jax.experimental import pallas as pl
from jax.experimental.pallas import tpu as pltpu
from jax.experimental.pallas import tpu_sc as plsc

_DIM = 64
_BATCH = 16384
_MARGIN = 4.0
_ALPHA = 0.01

_NW = 32
_BPW = _BATCH // _NW
_CHUNK = 128
_NCHUNK = _BPW // _CHUNK
_NGRP = _CHUNK // 16


def _sc_body(idx_all, ent_t, rel_t, nv_t, out,
             hp_b, rp_b, tp_b, vp_b, hn_b, rn_b, tn_b, vn_b,
             idx_v, out_v, sem):
  wid = lax.axis_index("s") * 2 + lax.axis_index("c")
  pltpu.sync_copy(idx_all.at[wid], idx_v)

  zero = jnp.zeros((16,), jnp.float32)
  hinge_acc = zero
  habs_acc = zero
  tabs_acc = zero
  rsq_acc = zero

  gathers = (
      (ent_t, 0, hp_b),
      (rel_t, 1, rp_b),
      (ent_t, 2, tp_b),
      (nv_t, 1, vp_b),
      (ent_t, 3, hn_b),
      (rel_t, 4, rn_b),
      (ent_t, 5, tn_b),
      (nv_t, 4, vn_b),
  )

  for c in range(_NCHUNK):
    def start_feat(f, _):
      for table, k, buf in gathers:
        pltpu.async_copy(table.at[f].at[idx_v.at[k, c]], buf.at[f], sem)
      return 0

    lax.fori_loop(0, _DIM, start_feat, 0)

    def wait_feat(f, _):
      for table, k, buf in gathers:
        pltpu.make_async_copy(table.at[f].at[idx_v.at[k, c]], buf.at[f],
                              sem).wait()
      return 0

    lax.fori_loop(0, _DIM, wait_feat, 0)

    def group_body(g, carry):
      hinge_a, habs_a, tabs_a, rsq_a = carry
      sl = pl.ds(pl.multiple_of(g * 16, 16), 16)

      def pass1(f, cr):
        cp_a, cn_a, ha, ta = cr
        hpv = hp_b[f, sl]
        tpv = tp_b[f, sl]
        vpv = vp_b[f, sl]
        hnv = hn_b[f, sl]
        tnv = tn_b[f, sl]
        vnv = vn_b[f, sl]
        cp_a = cp_a + (hpv - tpv) * vpv
        cn_a = cn_a + (hnv - tnv) * vnv
        ha = ha + jnp.abs(hpv) + jnp.abs(hnv)
        ta = ta + jnp.abs(tpv) + jnp.abs(tnv)
        return cp_a, cn_a, ha, ta

      cp_a, cn_a, habs_a, tabs_a = lax.fori_loop(
          0, _DIM, pass1, (zero, zero, habs_a, tabs_a))

      def pass2(f, cr):
        dp_a, dn_a, ra = cr
        hpv = hp_b[f, sl]
        tpv = tp_b[f, sl]
        vpv = vp_b[f, sl]
        rpv = rp_b[f, sl]
        hnv = hn_b[f, sl]
        tnv = tn_b[f, sl]
        vnv = vn_b[f, sl]
        rnv = rn_b[f, sl]
        sp = hpv - tpv + rpv - cp_a * vpv
        sn = hnv - tnv + rnv - cn_a * vnv
        dp_a = dp_a + jnp.abs(sp)
        dn_a = dn_a + jnp.abs(sn)
        ra = ra + rpv * rpv + rnv * rnv
        return dp_a, dn_a, ra

      dp_a, dn_a, rsq_a = lax.fori_loop(
          0, _DIM, pass2, (zero, zero, rsq_a))

      hinge_a = hinge_a + jnp.maximum(0.0, dp_a - dn_a + _MARGIN)
      return hinge_a, habs_a, tabs_a, rsq_a

    hinge_acc, habs_acc, tabs_acc, rsq_acc = lax.fori_loop(
        0, _NGRP, group_body, (hinge_acc, habs_acc, tabs_acc, rsq_acc))

  out_v[0, :] = hinge_acc
  out_v[1, :] = habs_acc
  out_v[2, :] = tabs_acc
  out_v[3, :] = rsq_acc
  pltpu.sync_copy(out_v, out.at[wid])


@jax.jit
def kernel(pos_triplets, neg_triplets, ent_emb, rel_emb, norm_vec):
  pos = pos_triplets.astype(jnp.int32)
  neg = neg_triplets.astype(jnp.int32)
  cols = jnp.concatenate([pos.T, neg.T], axis=0)
  idx_all = cols.reshape(6, _NW, _NCHUNK, _CHUNK).transpose(1, 0, 2, 3)

  call = pl.kernel(
      _sc_body,
      out_type=jax.ShapeDtypeStruct((_NW, 4, 16), jnp.float32),
      mesh=plsc.VectorSubcoreMesh(core_axis_name="c", subcore_axis_name="s"),
      scratch_types=[
          pltpu.VMEM((_DIM, _CHUNK), jnp.float32),
          pltpu.VMEM((_DIM, _CHUNK), jnp.float32),
          pltpu.VMEM((_DIM, _CHUNK), jnp.float32),
          pltpu.VMEM((_DIM, _CHUNK), jnp.float32),
          pltpu.VMEM((_DIM, _CHUNK), jnp.float32),
          pltpu.VMEM((_DIM, _CHUNK), jnp.float32),
          pltpu.VMEM((_DIM, _CHUNK), jnp.float32),
          pltpu.VMEM((_DIM, _CHUNK), jnp.float32),
          pltpu.VMEM((6, _NCHUNK, _CHUNK), jnp.int32),
          pltpu.VMEM((4, 16), jnp.float32),
          pltpu.SemaphoreType.DMA,
      ],
      compiler_params=pltpu.CompilerParams(use_tc_tiling_on_sc=False),
  )
  parts = call(idx_all, ent_emb.T, rel_emb.T, norm_vec.T)
  s = jnp.sum(parts, axis=(0, 2))
  loss = (s[0] / _BATCH
          + (_ALPHA / 3.0) * (s[1] / _BATCH + s[2] / _BATCH
                              + s[3] / (_BATCH * _DIM) - 4.0))
  return loss


# speedup vs baseline: 9.0725x; 9.0725x over previous
import jax
import jax.numpy as jnp
from jax import lax
from jax.experimental import pallas as pl
from jax.experimental.pallas import tpu as pltpu
from jax.experimental.pallas import tpu_sc as plsc

_DIM = 64
_BATCH = 16384
_MARGIN = 4.0
_ALPHA = 0.01

_NW = 32
_BPW = _BATCH // _NW
_CHUNK = 64
_NCHUNK = _BPW // _CHUNK
_NGRP = _CHUNK // 16


def _sc_body(idx_all, ent_p, rel_p, nv_p, out,
             hp_b, rp_b, tp_b, vp_b, hn_b, rn_b, tn_b, vn_b,
             idx_v, out_v, sem):
  wid = lax.axis_index("s") * 2 + lax.axis_index("c")
  pltpu.sync_copy(idx_all.at[wid], idx_v)

  zero = jnp.zeros((16,), jnp.float32)

  gathers = (
      (ent_p, 0, 6, hp_b),
      (rel_p, 1, 7, rp_b),
      (ent_p, 2, 8, tp_b),
      (nv_p, 1, 7, vp_b),
      (ent_p, 3, 9, hn_b),
      (rel_p, 4, 10, rn_b),
      (ent_p, 5, 11, tn_b),
      (nv_p, 4, 10, vn_b),
  )

  def chunk_body(c, acc):
    cps = [
        pltpu.async_copy(table.at[idx_v.at[k, c]], buf, sem)
        for table, k, _, buf in gathers
    ]
    for cp in cps:
      cp.wait()

    def group_body(g, carry):
      base = pl.multiple_of(g * 16, 16)
      offs = [idx_v[ko, c, pl.ds(base, 16)] for _, _, ko, _b in gathers]

      def elem(j, carry):
        hinge_a, habs_a, tabs_a, rsq_a = carry
        e = base + j

        def row(buf, ov):
          o = pl.multiple_of(ov[j], 16)
          return [buf[e, pl.ds(o + q * 16, 16)] for q in range(4)]

        hp = row(hp_b, offs[0])
        rp = row(rp_b, offs[1])
        tp = row(tp_b, offs[2])
        vp = row(vp_b, offs[3])
        hn = row(hn_b, offs[4])
        rn = row(rn_b, offs[5])
        tn = row(tn_b, offs[6])
        vn = row(vn_b, offs[7])

        dp = [hp[q] - tp[q] for q in range(4)]
        dn = [hn[q] - tn[q] for q in range(4)]
        cp_s = jnp.sum(sum([dp[q] * vp[q] for q in range(4)], zero))
        cn_s = jnp.sum(sum([dn[q] * vn[q] for q in range(4)], zero))

        sp = [dp[q] + rp[q] - cp_s * vp[q] for q in range(4)]
        sn = [dn[q] + rn[q] - cn_s * vn[q] for q in range(4)]
        pd = jnp.sum(sum([jnp.abs(sp[q]) for q in range(4)], zero))
        nd = jnp.sum(sum([jnp.abs(sn[q]) for q in range(4)], zero))
        hinge_a = hinge_a + jnp.maximum(jnp.float32(0.0), pd - nd + _MARGIN)

        habs_a = habs_a + sum(
            [jnp.abs(hp[q]) + jnp.abs(hn[q]) for q in range(4)], zero)
        tabs_a = tabs_a + sum(
            [jnp.abs(tp[q]) + jnp.abs(tn[q]) for q in range(4)], zero)
        rsq_a = rsq_a + sum(
            [rp[q] * rp[q] + rn[q] * rn[q] for q in range(4)], zero)
        return hinge_a, habs_a, tabs_a, rsq_a

      for j in range(16):
        carry = elem(j, carry)
      return carry

    return lax.fori_loop(0, _NGRP, group_body, acc)

  hinge_acc, habs_acc, tabs_acc, rsq_acc = lax.fori_loop(
      0, _NCHUNK, chunk_body, (jnp.float32(0.0), zero, zero, zero))

  out_v[0, :] = habs_acc
  out_v[1, :] = tabs_acc
  out_v[2, :] = rsq_acc
  out_v[3, :] = zero + hinge_acc * 0.0625
  pltpu.sync_copy(out_v, out.at[wid])


@jax.jit
def kernel(pos_triplets, neg_triplets, ent_emb, rel_emb, norm_vec):
  pos = pos_triplets.astype(jnp.int32)
  neg = neg_triplets.astype(jnp.int32)
  cols = jnp.concatenate([pos.T, neg.T], axis=0)
  rows = jnp.concatenate([cols >> 1, (cols & 1) * _DIM], axis=0)
  idx_all = rows.reshape(12, _NW, _NCHUNK, _CHUNK).transpose(1, 0, 2, 3)

  call = pl.kernel(
      _sc_body,
      out_type=jax.ShapeDtypeStruct((_NW, 4, 16), jnp.float32),
      mesh=plsc.VectorSubcoreMesh(core_axis_name="c", subcore_axis_name="s"),
      scratch_types=[
          pltpu.VMEM((_CHUNK, 2 * _DIM), jnp.float32),
          pltpu.VMEM((_CHUNK, 2 * _DIM), jnp.float32),
          pltpu.VMEM((_CHUNK, 2 * _DIM), jnp.float32),
          pltpu.VMEM((_CHUNK, 2 * _DIM), jnp.float32),
          pltpu.VMEM((_CHUNK, 2 * _DIM), jnp.float32),
          pltpu.VMEM((_CHUNK, 2 * _DIM), jnp.float32),
          pltpu.VMEM((_CHUNK, 2 * _DIM), jnp.float32),
          pltpu.VMEM((_CHUNK, 2 * _DIM), jnp.float32),
          pltpu.VMEM((12, _NCHUNK, _CHUNK), jnp.int32),
          pltpu.VMEM((4, 16), jnp.float32),
          pltpu.SemaphoreType.DMA,
      ],
      compiler_params=pltpu.CompilerParams(needs_layout_passes=False),
  )
  parts = call(idx_all,
               ent_emb.reshape(-1, 2 * _DIM),
               rel_emb.reshape(-1, 2 * _DIM),
               norm_vec.reshape(-1, 2 * _DIM))
  s = jnp.sum(parts, axis=(0, 2))
  loss = (s[3] / _BATCH
          + (_ALPHA / 3.0) * (s[0] / _BATCH + s[1] / _BATCH
                              + s[2] / (_BATCH * _DIM) - 4.0))
  return loss


# speedup vs baseline: 21.7480x; 2.3971x over previous
import jax
import jax.numpy as jnp
from jax import lax
from jax.experimental import pallas as pl
from jax.experimental.pallas import tpu as pltpu
from jax.experimental.pallas import tpu_sc as plsc

_DIM = 64
_BATCH = 16384
_MARGIN = 4.0
_ALPHA = 0.01

_NW = 32
_BPW = _BATCH // _NW
_CHUNK = 64
_NCHUNK = _BPW // _CHUNK
_NGRP = _CHUNK // 16

_V = 1000000
_PW = 16384
_PNB = 31
_PH = _PNB * _PW
_PLASTB = (_V - 1) // _PW


def _pack_body(a_ref, b_ref, o_ref):
  o_ref[:, 0:_DIM] = a_ref[...].T
  o_ref[:, _DIM:2 * _DIM] = b_ref[...].T


def _pack_table(t):
  tt = t.T
  return pl.pallas_call(
      _pack_body,
      grid=(_PNB,),
      in_specs=[
          pl.BlockSpec((_DIM, _PW), lambda i: (0, i)),
          pl.BlockSpec((_DIM, _PW),
                       lambda i: (0, jnp.minimum(i + _PNB, _PLASTB))),
      ],
      out_specs=pl.BlockSpec((_PW, 2 * _DIM), lambda i: (i, 0)),
      out_shape=jax.ShapeDtypeStruct((_PH, 2 * _DIM), jnp.float32),
      compiler_params=pltpu.CompilerParams(
          dimension_semantics=("arbitrary",)),
  )(tt, tt)


def _row(buf, e, o):
  return [buf[e, pl.ds(o + q * 16, 16)] for q in range(4)]


def _sc1_body(idx_all, ent_p, dstage, part, hp_b, tp_b, hn_b, tn_b,
              idx_v, dv, out_v, sem):
  wid = lax.axis_index("s") * 2 + lax.axis_index("c")
  pltpu.sync_copy(idx_all.at[wid], idx_v)
  zero = jnp.zeros((16,), jnp.float32)

  def chunk_body(c, acc):
    cps = [
        pltpu.async_copy(ent_p.at[idx_v.at[0, c]], hp_b, sem),
        pltpu.async_copy(ent_p.at[idx_v.at[2, c]], tp_b, sem),
        pltpu.async_copy(ent_p.at[idx_v.at[4, c]], hn_b, sem),
        pltpu.async_copy(ent_p.at[idx_v.at[6, c]], tn_b, sem),
    ]
    for cp in cps:
      cp.wait()

    def group_body(g, carry):
      base = pl.multiple_of(g * 16, 16)
      offs = [idx_v[k, c, pl.ds(base, 16)] for k in (1, 3, 5, 7)]

      def elem(j, carry):
        habs_a, tabs_a = carry
        e = base + j
        hp = _row(hp_b, e, pl.multiple_of(offs[0][j], 16))
        tp = _row(tp_b, e, pl.multiple_of(offs[1][j], 16))
        hn = _row(hn_b, e, pl.multiple_of(offs[2][j], 16))
        tn = _row(tn_b, e, pl.multiple_of(offs[3][j], 16))
        for q in range(4):
          dv[e, pl.ds(q * 16, 16)] = hp[q] - tp[q]
          dv[e, pl.ds(_DIM + q * 16, 16)] = hn[q] - tn[q]
        habs_a = habs_a + sum(
            [jnp.abs(hp[q]) + jnp.abs(hn[q]) for q in range(4)], zero)
        tabs_a = tabs_a + sum(
            [jnp.abs(tp[q]) + jnp.abs(tn[q]) for q in range(4)], zero)
        return habs_a, tabs_a

      for j in range(16):
        carry = elem(j, carry)
      return carry

    acc = lax.fori_loop(0, _NGRP, group_body, acc)
    pltpu.sync_copy(dv, dstage.at[pl.ds(wid * _BPW + c * _CHUNK, _CHUNK)])
    return acc

  habs_acc, tabs_acc = lax.fori_loop(0, _NCHUNK, chunk_body, (zero, zero))
  out_v[0, :] = habs_acc
  out_v[1, :] = tabs_acc
  pltpu.sync_copy(out_v, part.at[wid])


def _sc2_body(idx_all, rel_p, nv_p, dstage, part,
              rp_b, vp_b, rn_b, vn_b, idx_v, dv, out_v, sem):
  wid = lax.axis_index("s") * 2 + lax.axis_index("c")
  pltpu.sync_copy(idx_all.at[wid], idx_v)
  zero = jnp.zeros((16,), jnp.float32)

  def chunk_body(c, acc):
    cps = [
        pltpu.async_copy(rel_p.at[idx_v.at[0, c]], rp_b, sem),
        pltpu.async_copy(nv_p.at[idx_v.at[0, c]], vp_b, sem),
        pltpu.async_copy(rel_p.at[idx_v.at[2, c]], rn_b, sem),
        pltpu.async_copy(nv_p.at[idx_v.at[2, c]], vn_b, sem),
    ]
    pltpu.sync_copy(dstage.at[pl.ds(wid * _BPW + c * _CHUNK, _CHUNK)], dv)
    for cp in cps:
      cp.wait()

    def group_body(g, carry):
      base = pl.multiple_of(g * 16, 16)
      offs = [idx_v[k, c, pl.ds(base, 16)] for k in (1, 3)]

      def elem(j, carry):
        hinge_a, rsq_a = carry
        e = base + j
        op = pl.multiple_of(offs[0][j], 16)
        on = pl.multiple_of(offs[1][j], 16)
        dp = _row(dv, e, 0)
        dn = _row(dv, e, _DIM)
        rp = _row(rp_b, e, op)
        vp = _row(vp_b, e, op)
        rn = _row(rn_b, e, on)
        vn = _row(vn_b, e, on)

        cp_s = jnp.sum(sum([dp[q] * vp[q] for q in range(4)], zero))
        cn_s = jnp.sum(sum([dn[q] * vn[q] for q in range(4)], zero))
        sp = [dp[q] + rp[q] - cp_s * vp[q] for q in range(4)]
        sn = [dn[q] + rn[q] - cn_s * vn[q] for q in range(4)]
        pd = jnp.sum(sum([jnp.abs(sp[q]) for q in range(4)], zero))
        nd = jnp.sum(sum([jnp.abs(sn[q]) for q in range(4)], zero))
        hinge_a = hinge_a + jnp.maximum(jnp.float32(0.0), pd - nd + _MARGIN)
        rsq_a = rsq_a + sum(
            [rp[q] * rp[q] + rn[q] * rn[q] for q in range(4)], zero)
        return hinge_a, rsq_a

      for j in range(16):
        carry = elem(j, carry)
      return carry

    return lax.fori_loop(0, _NGRP, group_body, acc)

  hinge_acc, rsq_acc = lax.fori_loop(
      0, _NCHUNK, chunk_body, (jnp.float32(0.0), zero))
  out_v[0, :] = rsq_acc
  out_v[1, :] = zero + hinge_acc * 0.0625
  pltpu.sync_copy(out_v, part.at[wid])


@jax.jit
def kernel(pos_triplets, neg_triplets, ent_emb, rel_emb, norm_vec):
  pos = pos_triplets.astype(jnp.int32)
  neg = neg_triplets.astype(jnp.int32)
  cols = jnp.concatenate([pos.T, neg.T], axis=0)
  hi = (cols >= _PH).astype(jnp.int32)
  rows = cols - hi * _PH
  offs = hi * _DIM

  def worker_layout(lists):
    arr = jnp.stack(lists)
    k = arr.shape[0]
    return arr.reshape(k, _NW, _NCHUNK, _CHUNK).transpose(1, 0, 2, 3)

  idx1 = worker_layout([rows[0], offs[0], rows[2], offs[2],
                        rows[3], offs[3], rows[5], offs[5]])
  idx2 = worker_layout([rows[1], offs[1], rows[4], offs[4]])

  mesh = plsc.VectorSubcoreMesh(core_axis_name="c", subcore_axis_name="s")
  cparams = pltpu.CompilerParams(needs_layout_passes=False)

  sc1 = pl.kernel(
      _sc1_body,
      out_type=(jax.ShapeDtypeStruct((_BATCH, 2 * _DIM), jnp.float32),
                jax.ShapeDtypeStruct((_NW, 2, 16), jnp.float32)),
      mesh=mesh,
      scratch_types=[
          pltpu.VMEM((_CHUNK, 2 * _DIM), jnp.float32),
          pltpu.VMEM((_CHUNK, 2 * _DIM), jnp.float32),
          pltpu.VMEM((_CHUNK, 2 * _DIM), jnp.float32),
          pltpu.VMEM((_CHUNK, 2 * _DIM), jnp.float32),
          pltpu.VMEM((8, _NCHUNK, _CHUNK), jnp.int32),
          pltpu.VMEM((_CHUNK, 2 * _DIM), jnp.float32),
          pltpu.VMEM((2, 16), jnp.float32),
          pltpu.SemaphoreType.DMA,
      ],
      compiler_params=cparams,
  )

  sc2 = pl.kernel(
      _sc2_body,
      out_type=jax.ShapeDtypeStruct((_NW, 2, 16), jnp.float32),
      mesh=mesh,
      scratch_types=[
          pltpu.VMEM((_CHUNK, 2 * _DIM), jnp.float32),
          pltpu.VMEM((_CHUNK, 2 * _DIM), jnp.float32),
          pltpu.VMEM((_CHUNK, 2 * _DIM), jnp.float32),
          pltpu.VMEM((_CHUNK, 2 * _DIM), jnp.float32),
          pltpu.VMEM((4, _NCHUNK, _CHUNK), jnp.int32),
          pltpu.VMEM((_CHUNK, 2 * _DIM), jnp.float32),
          pltpu.VMEM((2, 16), jnp.float32),
          pltpu.SemaphoreType.DMA,
      ],
      compiler_params=cparams,
  )

  ent_p = _pack_table(ent_emb)
  rel_p = _pack_table(rel_emb)
  nv_p = _pack_table(norm_vec)

  dstage, part1 = sc1(idx1, ent_p)
  part2 = sc2(idx2, rel_p, nv_p, dstage)

  s1 = jnp.sum(part1, axis=(0, 2))
  s2 = jnp.sum(part2, axis=(0, 2))
  loss = (s2[1] / _BATCH
          + (_ALPHA / 3.0) * (s1[0] / _BATCH + s1[1] / _BATCH
                              + s2[0] / (_BATCH * _DIM) - 4.0))
  return loss


# speedup vs baseline: 21.8791x; 1.0060x over previous
import jax
import jax.numpy as jnp
from jax import lax
from jax.experimental import pallas as pl
from jax.experimental.pallas import tpu as pltpu
from jax.experimental.pallas import tpu_sc as plsc

_DIM = 64
_BATCH = 16384
_MARGIN = 4.0
_ALPHA = 0.01

_NW = 32
_BPW = _BATCH // _NW
_CHUNK = 128
_NCHUNK = _BPW // _CHUNK
_NGRP = _CHUNK // 16

_V = 1000000
_PW = 16384
_PNB = 31
_PH = _PNB * _PW
_PLASTB = (_V - 1) // _PW


def _pack_body(a_ref, b_ref, o_ref):
  o_ref[:, 0:_DIM] = a_ref[...].T
  o_ref[:, _DIM:2 * _DIM] = b_ref[...].T


def _pack_table(t):
  tt = t.T
  return pl.pallas_call(
      _pack_body,
      grid=(_PNB,),
      in_specs=[
          pl.BlockSpec((_DIM, _PW), lambda i: (0, i)),
          pl.BlockSpec((_DIM, _PW),
                       lambda i: (0, jnp.minimum(i + _PNB, _PLASTB))),
      ],
      out_specs=pl.BlockSpec((_PW, 2 * _DIM), lambda i: (i, 0)),
      out_shape=jax.ShapeDtypeStruct((_PH, 2 * _DIM), jnp.float32),
      compiler_params=pltpu.CompilerParams(
          dimension_semantics=("arbitrary",)),
  )(tt, tt)


def _row(buf, e, o):
  return [buf[e, pl.ds(o + q * 16, 16)] for q in range(4)]


def _sc1_body(idx_all, ent_p, dstage, part, hp_b, tp_b, hn_b, tn_b,
              idx_v, dv, out_v, sem):
  wid = lax.axis_index("s") * 2 + lax.axis_index("c")
  pltpu.sync_copy(idx_all.at[wid], idx_v)
  zero = jnp.zeros((16,), jnp.float32)

  def chunk_body(c, acc):
    cps = [
        pltpu.async_copy(ent_p.at[idx_v.at[0, c]], hp_b, sem),
        pltpu.async_copy(ent_p.at[idx_v.at[2, c]], tp_b, sem),
        pltpu.async_copy(ent_p.at[idx_v.at[4, c]], hn_b, sem),
        pltpu.async_copy(ent_p.at[idx_v.at[6, c]], tn_b, sem),
    ]
    for cp in cps:
      cp.wait()

    def group_body(g, carry):
      base = pl.multiple_of(g * 16, 16)
      offs = [idx_v[k, c, pl.ds(base, 16)] for k in (1, 3, 5, 7)]

      def elem(j, carry):
        habs_a, tabs_a = carry
        e = base + j
        hp = _row(hp_b, e, pl.multiple_of(offs[0][j], 16))
        tp = _row(tp_b, e, pl.multiple_of(offs[1][j], 16))
        hn = _row(hn_b, e, pl.multiple_of(offs[2][j], 16))
        tn = _row(tn_b, e, pl.multiple_of(offs[3][j], 16))
        for q in range(4):
          dv[e, pl.ds(q * 16, 16)] = hp[q] - tp[q]
          dv[e, pl.ds(_DIM + q * 16, 16)] = hn[q] - tn[q]
        habs_a = habs_a + sum(
            [jnp.abs(hp[q]) + jnp.abs(hn[q]) for q in range(4)], zero)
        tabs_a = tabs_a + sum(
            [jnp.abs(tp[q]) + jnp.abs(tn[q]) for q in range(4)], zero)
        return habs_a, tabs_a

      for j in range(16):
        carry = elem(j, carry)
      return carry

    acc = lax.fori_loop(0, _NGRP, group_body, acc)
    pltpu.sync_copy(dv, dstage.at[pl.ds(wid * _BPW + c * _CHUNK, _CHUNK)])
    return acc

  habs_acc, tabs_acc = lax.fori_loop(0, _NCHUNK, chunk_body, (zero, zero))
  out_v[0, :] = habs_acc
  out_v[1, :] = tabs_acc
  pltpu.sync_copy(out_v, part.at[wid])


def _sc2_body(idx_all, rel_p, nv_p, dstage, part,
              rp_b, vp_b, rn_b, vn_b, idx_v, dv, out_v, sem):
  wid = lax.axis_index("s") * 2 + lax.axis_index("c")
  pltpu.sync_copy(idx_all.at[wid], idx_v)
  zero = jnp.zeros((16,), jnp.float32)

  def chunk_body(c, acc):
    cps = [
        pltpu.async_copy(rel_p.at[idx_v.at[0, c]], rp_b, sem),
        pltpu.async_copy(nv_p.at[idx_v.at[0, c]], vp_b, sem),
        pltpu.async_copy(rel_p.at[idx_v.at[2, c]], rn_b, sem),
        pltpu.async_copy(nv_p.at[idx_v.at[2, c]], vn_b, sem),
    ]
    pltpu.sync_copy(dstage.at[pl.ds(wid * _BPW + c * _CHUNK, _CHUNK)], dv)
    for cp in cps:
      cp.wait()

    def group_body(g, carry):
      base = pl.multiple_of(g * 16, 16)
      offs = [idx_v[k, c, pl.ds(base, 16)] for k in (1, 3)]

      def elem(j, carry):
        hinge_a, rsq_a = carry
        e = base + j
        op = pl.multiple_of(offs[0][j], 16)
        on = pl.multiple_of(offs[1][j], 16)
        dp = _row(dv, e, 0)
        dn = _row(dv, e, _DIM)
        rp = _row(rp_b, e, op)
        vp = _row(vp_b, e, op)
        rn = _row(rn_b, e, on)
        vn = _row(vn_b, e, on)

        cp_s = jnp.sum(sum([dp[q] * vp[q] for q in range(4)], zero))
        cn_s = jnp.sum(sum([dn[q] * vn[q] for q in range(4)], zero))
        sp = [dp[q] + rp[q] - cp_s * vp[q] for q in range(4)]
        sn = [dn[q] + rn[q] - cn_s * vn[q] for q in range(4)]
        pd = jnp.sum(sum([jnp.abs(sp[q]) for q in range(4)], zero))
        nd = jnp.sum(sum([jnp.abs(sn[q]) for q in range(4)], zero))
        hinge_a = hinge_a + jnp.maximum(jnp.float32(0.0), pd - nd + _MARGIN)
        rsq_a = rsq_a + sum(
            [rp[q] * rp[q] + rn[q] * rn[q] for q in range(4)], zero)
        return hinge_a, rsq_a

      for j in range(16):
        carry = elem(j, carry)
      return carry

    return lax.fori_loop(0, _NGRP, group_body, acc)

  hinge_acc, rsq_acc = lax.fori_loop(
      0, _NCHUNK, chunk_body, (jnp.float32(0.0), zero))
  out_v[0, :] = rsq_acc
  out_v[1, :] = zero + hinge_acc * 0.0625
  pltpu.sync_copy(out_v, part.at[wid])


@jax.jit
def kernel(pos_triplets, neg_triplets, ent_emb, rel_emb, norm_vec):
  pos = pos_triplets.astype(jnp.int32)
  neg = neg_triplets.astype(jnp.int32)
  cols = jnp.concatenate([pos.T, neg.T], axis=0)
  hi = (cols >= _PH).astype(jnp.int32)
  rows = cols - hi * _PH
  offs = hi * _DIM

  def worker_layout(lists):
    arr = jnp.stack(lists)
    k = arr.shape[0]
    return arr.reshape(k, _NW, _NCHUNK, _CHUNK).transpose(1, 0, 2, 3)

  idx1 = worker_layout([rows[0], offs[0], rows[2], offs[2],
                        rows[3], offs[3], rows[5], offs[5]])
  idx2 = worker_layout([rows[1], offs[1], rows[4], offs[4]])

  mesh = plsc.VectorSubcoreMesh(core_axis_name="c", subcore_axis_name="s")
  cparams = pltpu.CompilerParams(needs_layout_passes=False)

  sc1 = pl.kernel(
      _sc1_body,
      out_type=(jax.ShapeDtypeStruct((_BATCH, 2 * _DIM), jnp.float32),
                jax.ShapeDtypeStruct((_NW, 2, 16), jnp.float32)),
      mesh=mesh,
      scratch_types=[
          pltpu.VMEM((_CHUNK, 2 * _DIM), jnp.float32),
          pltpu.VMEM((_CHUNK, 2 * _DIM), jnp.float32),
          pltpu.VMEM((_CHUNK, 2 * _DIM), jnp.float32),
          pltpu.VMEM((_CHUNK, 2 * _DIM), jnp.float32),
          pltpu.VMEM((8, _NCHUNK, _CHUNK), jnp.int32),
          pltpu.VMEM((_CHUNK, 2 * _DIM), jnp.float32),
          pltpu.VMEM((2, 16), jnp.float32),
          pltpu.SemaphoreType.DMA,
      ],
      compiler_params=cparams,
  )

  sc2 = pl.kernel(
      _sc2_body,
      out_type=jax.ShapeDtypeStruct((_NW, 2, 16), jnp.float32),
      mesh=mesh,
      scratch_types=[
          pltpu.VMEM((_CHUNK, 2 * _DIM), jnp.float32),
          pltpu.VMEM((_CHUNK, 2 * _DIM), jnp.float32),
          pltpu.VMEM((_CHUNK, 2 * _DIM), jnp.float32),
          pltpu.VMEM((_CHUNK, 2 * _DIM), jnp.float32),
          pltpu.VMEM((4, _NCHUNK, _CHUNK), jnp.int32),
          pltpu.VMEM((_CHUNK, 2 * _DIM), jnp.float32),
          pltpu.VMEM((2, 16), jnp.float32),
          pltpu.SemaphoreType.DMA,
      ],
      compiler_params=cparams,
  )

  ent_p = _pack_table(ent_emb)
  rel_p = _pack_table(rel_emb)
  nv_p = _pack_table(norm_vec)

  dstage, part1 = sc1(idx1, ent_p)
  part2 = sc2(idx2, rel_p, nv_p, dstage)

  s1 = jnp.sum(part1, axis=(0, 2))
  s2 = jnp.sum(part2, axis=(0, 2))
  loss = (s2[1] / _BATCH
          + (_ALPHA / 3.0) * (s1[0] / _BATCH + s1[1] / _BATCH
                              + s2[0] / (_BATCH * _DIM) - 4.0))
  return loss
